# baseline (speedup 1.0000x reference)
import functools
import math

import jax
import jax.numpy as jnp
from jax.experimental import pallas as pl
from jax.experimental.pallas import tpu as pltpu

N_EXP = 8
K_TOP = 2
_INV_SQRT2 = 1.0 / math.sqrt(2.0)


def _gelu_exact(x):
    return 0.5 * x * (1.0 + jax.lax.erf(x * _INV_SQRT2))


def _router_body(x_ref, wg_ref, logits_ref, sel_ref, wdense_ref, mask_ref):
    x = x_ref[...]
    wg = wg_ref[...]
    logits = jax.lax.dot_general(
        x, wg, (((1,), (1,)), ((), ())),
        preferred_element_type=jnp.float32,
        precision=jax.lax.Precision.DEFAULT)
    logits_ref[...] = logits
    n = logits.shape[0]
    iota = jax.lax.broadcasted_iota(jnp.int32, (n, N_EXP), 1)
    m1 = jnp.max(logits, axis=1, keepdims=True)
    i1 = jnp.min(jnp.where(logits == m1, iota, N_EXP), axis=1, keepdims=True)
    masked = jnp.where(iota == i1, -jnp.inf, logits)
    m2 = jnp.max(masked, axis=1, keepdims=True)
    i2 = jnp.min(jnp.where(masked == m2, iota, N_EXP), axis=1, keepdims=True)
    e2 = jnp.exp(m2 - m1)
    w1 = 1.0 / (1.0 + e2)
    w2 = e2 * w1
    k_iota = jax.lax.broadcasted_iota(jnp.int32, (n, K_TOP), 1)
    sel_ref[...] = jnp.where(k_iota == 0, i1, i2)
    sel1 = (iota == i1).astype(jnp.float32)
    sel2 = (iota == i2).astype(jnp.float32)
    wdense_ref[...] = sel1 * w1 + sel2 * w2
    mask_ref[...] = sel1 + sel2


def _ffn_body(x_ref, w1_ref, w2_ref, wdense_ref, mask_ref,
              full_ref, fin_ref, acc_ref, fin_acc_ref, *, bt, prec):
    e = pl.program_id(0)
    t = pl.program_id(1)
    k = pl.program_id(2)
    nk = pl.num_programs(2)
    x = x_ref[...]
    w1 = w1_ref[0]
    h = jax.lax.dot_general(x, w1, (((1,), (1,)), ((), ())),
                            preferred_element_type=jnp.float32, precision=prec)
    h = _gelu_exact(h)
    w2 = w2_ref[0]
    o = jax.lax.dot_general(h, w2, (((1,), (1,)), ((), ())),
                            preferred_element_type=jnp.float32, precision=prec)

    @pl.when(k == 0)
    def _():
        acc_ref[...] = o

    @pl.when(k > 0)
    def _():
        acc_ref[...] += o

    @pl.when(k == nk - 1)
    def _():
        acc = acc_ref[...]
        lane = jax.lax.broadcasted_iota(jnp.int32, (bt, N_EXP), 1)
        one_e = (lane == e).astype(jnp.float32)
        m_col = jnp.sum(mask_ref[...] * one_e, axis=1, keepdims=True)
        w_col = jnp.sum(wdense_ref[...] * one_e, axis=1, keepdims=True)
        full_ref[...] = acc * m_col
        sl = pl.ds(t * bt, bt)

        @pl.when(e == 0)
        def _():
            fin_acc_ref[sl, :] = acc * w_col

        @pl.when(e > 0)
        def _():
            fin_acc_ref[sl, :] += acc * w_col

        @pl.when(e == N_EXP - 1)
        def _():
            fin_ref[...] = fin_acc_ref[sl, :]


def kernel(hidden_states, W_gate, W1, W2):
    B, T, C = hidden_states.shape
    E, INTER, _ = W1.shape
    N = B * T
    flat = hidden_states.reshape(N, C)

    logits, sel, wdense, mask = pl.pallas_call(
        _router_body,
        out_shape=[
            jax.ShapeDtypeStruct((N, N_EXP), jnp.float32),
            jax.ShapeDtypeStruct((N, K_TOP), jnp.int32),
            jax.ShapeDtypeStruct((N, N_EXP), jnp.float32),
            jax.ShapeDtypeStruct((N, N_EXP), jnp.float32),
        ],
    )(flat, W_gate)

    BT = min(512, N)
    IC = min(768, INTER)
    grid = (E, N // BT, INTER // IC)
    full2d, fin = pl.pallas_call(
        functools.partial(_ffn_body, bt=BT, prec=jax.lax.Precision.HIGHEST),
        grid=grid,
        in_specs=[
            pl.BlockSpec((BT, C), lambda e, t, k: (t, 0)),
            pl.BlockSpec((1, IC, C), lambda e, t, k: (e, k, 0)),
            pl.BlockSpec((1, C, IC), lambda e, t, k: (e, 0, k)),
            pl.BlockSpec((BT, N_EXP), lambda e, t, k: (t, 0)),
            pl.BlockSpec((BT, N_EXP), lambda e, t, k: (t, 0)),
        ],
        out_specs=[
            pl.BlockSpec((BT, C), lambda e, t, k: (t, e)),
            pl.BlockSpec((BT, C), lambda e, t, k: (t, 0)),
        ],
        out_shape=[
            jax.ShapeDtypeStruct((N, E * C), jnp.float32),
            jax.ShapeDtypeStruct((N, C), jnp.float32),
        ],
        scratch_shapes=[
            pltpu.VMEM((BT, C), jnp.float32),
            pltpu.VMEM((N, C), jnp.float32),
        ],
    )(flat, W1, W2, wdense, mask)

    return (fin.reshape(B, T, C), full2d.reshape(N, E, C), logits, sel)
